# Initial kernel scaffold; baseline (speedup 1.0000x reference)
import jax
import jax.numpy as jnp
from jax.experimental import pallas as pl


def kernel(x, edge_index, Wl1, bl1, Wr1, Wl2, bl2, Wr2):
    raise NotImplementedError("write your pallas kernel here")


# speedup vs baseline: 4.7276x; 4.7276x over previous
import functools

import jax
import jax.numpy as jnp
from jax import lax
from jax.experimental import pallas as pl
from jax.experimental.pallas import tpu as pltpu
from jax.experimental.pallas import tpu_sc as plsc

N_NODES = 10000
N_EDGES = 320000
D_IN = 128
D_HID = 128
D_OUT = 64

N_TILES = 16
N_PAD = 10112
ROWS_PER_TILE = N_PAD // N_TILES
STEP = 128
S_T = 160
E_PAD = N_TILES * S_T * STEP
DEG_SPLIT = (S_T + 1) // 2
ROW_BLK = 1000


def _make_sc_aggregate(width, with_deg):
  out_type = [jax.ShapeDtypeStruct((N_PAD, width), jnp.float32),
              jax.ShapeDtypeStruct((N_PAD, width), jnp.float32)]
  scratch = [pltpu.VMEM((S_T, STEP), jnp.int32),
             pltpu.VMEM((S_T, STEP), jnp.int32),
             pltpu.VMEM((STEP, width), jnp.float32),
             pltpu.VMEM_SHARED((N_PAD, width), jnp.float32),
             pltpu.SemaphoreType.DMA]
  if with_deg:
    out_type += [jax.ShapeDtypeStruct((N_PAD, 16), jnp.float32),
                 jax.ShapeDtypeStruct((N_PAD, 16), jnp.float32)]
    scratch += [pltpu.VMEM((STEP, 16), jnp.float32),
                pltpu.VMEM_SHARED((N_PAD, 16), jnp.float32)]

  mesh = plsc.VectorSubcoreMesh(core_axis_name="c", subcore_axis_name="s")

  def body(*refs):
    if with_deg:
      (ya, yb, src_hbm, dst_hbm, zf_hbm, zd_hbm, ones_hbm,
       out_a, out_b, deg_a, deg_b,
       src_v, dst_v, rows_v, agg_sh, sem, ones_v, deg_sh) = refs
    else:
      (ya, yb, src_hbm, dst_hbm, zf_hbm,
       out_a, out_b,
       src_v, dst_v, rows_v, agg_sh, sem) = refs
    c = lax.axis_index("c")
    s = lax.axis_index("s")
    r0 = s * ROWS_PER_TILE
    tile_rows = pl.ds(r0, ROWS_PER_TILE)

    pltpu.sync_copy(zf_hbm.at[tile_rows], agg_sh.at[tile_rows])
    if with_deg:
      pltpu.sync_copy(zd_hbm.at[tile_rows], deg_sh.at[tile_rows])
      pltpu.sync_copy(ones_hbm, ones_v)
    pltpu.sync_copy(src_hbm.at[pl.ds(s * S_T, S_T)], src_v)
    pltpu.sync_copy(dst_hbm.at[pl.ds(s * S_T, S_T)], dst_v)
    plsc.subcore_barrier()

    def step(j, carry):
      @pl.when(c == 0)
      def _():
        pltpu.async_copy(ya.at[src_v.at[j]], rows_v, sem).wait()

      @pl.when(c == 1)
      def _():
        pltpu.async_copy(yb.at[src_v.at[j]], rows_v, sem).wait()

      pltpu.sync_copy(rows_v, agg_sh.at[dst_v.at[j]], add=True)
      if with_deg:
        count_here = jnp.where(c == 0, j < DEG_SPLIT, j >= DEG_SPLIT)

        @pl.when(count_here)
        def _():
          pltpu.sync_copy(ones_v, deg_sh.at[dst_v.at[j]], add=True)
      return carry

    lax.fori_loop(0, S_T, step, 0)
    plsc.subcore_barrier()

    @pl.when(c == 0)
    def _():
      pltpu.sync_copy(agg_sh.at[tile_rows], out_a.at[tile_rows])
      if with_deg:
        pltpu.sync_copy(deg_sh.at[tile_rows], deg_a.at[tile_rows])

    @pl.when(c == 1)
    def _():
      pltpu.sync_copy(agg_sh.at[tile_rows], out_b.at[tile_rows])
      if with_deg:
        pltpu.sync_copy(deg_sh.at[tile_rows], deg_b.at[tile_rows])

  return pl.kernel(body, out_type=out_type, mesh=mesh, scratch_types=scratch,
                   compiler_params=pltpu.CompilerParams(use_tc_tiling_on_sc=False))


def _tc_layer1(x, wcat):
  def body(x_ref, w_ref, ya_ref, yb_ref, z_ref):
    y = jnp.dot(x_ref[...], w_ref[...], preferred_element_type=jnp.float32)
    ya_ref[...] = y[:, :64]
    yb_ref[...] = y[:, 64:128]
    z_ref[...] = y[:, 128:]

  return pl.pallas_call(
      body,
      grid=(N_NODES // ROW_BLK,),
      in_specs=[pl.BlockSpec((ROW_BLK, D_IN), lambda i: (i, 0)),
                pl.BlockSpec((D_IN, 2 * D_HID), lambda i: (0, 0))],
      out_specs=[pl.BlockSpec((ROW_BLK, 64), lambda i: (i, 0)),
                 pl.BlockSpec((ROW_BLK, 64), lambda i: (i, 0)),
                 pl.BlockSpec((ROW_BLK, D_HID), lambda i: (i, 0))],
      out_shape=[jax.ShapeDtypeStruct((N_NODES, 64), jnp.float32),
                 jax.ShapeDtypeStruct((N_NODES, 64), jnp.float32),
                 jax.ShapeDtypeStruct((N_NODES, D_HID), jnp.float32)],
  )(x, wcat)


def _tc_layer2(agg_a, agg_b, deg_a, deg_b, z1, bl1, wcat, bl2):
  def body(a_ref, b_ref, da_ref, db_ref, z1_ref, bl1_ref, w_ref, bl2_ref,
           ya_ref, yb_ref, z2_ref):
    deg = da_ref[:, 0:1] + db_ref[:, 0:1]
    inv = 1.0 / jnp.maximum(deg, 1.0)
    mean = jnp.concatenate([a_ref[...], b_ref[...]], axis=1) * inv
    h = jnp.maximum(mean + bl1_ref[...] + z1_ref[...], 0.0)
    y2 = jnp.dot(h, w_ref[...], preferred_element_type=jnp.float32)
    ya_ref[...] = y2[:, :32]
    yb_ref[...] = y2[:, 32:64]
    z2_ref[...] = y2[:, 64:] + bl2_ref[...]

  return pl.pallas_call(
      body,
      grid=(N_NODES // ROW_BLK,),
      in_specs=[pl.BlockSpec((ROW_BLK, 64), lambda i: (i, 0)),
                pl.BlockSpec((ROW_BLK, 64), lambda i: (i, 0)),
                pl.BlockSpec((ROW_BLK, 16), lambda i: (i, 0)),
                pl.BlockSpec((ROW_BLK, 16), lambda i: (i, 0)),
                pl.BlockSpec((ROW_BLK, D_HID), lambda i: (i, 0)),
                pl.BlockSpec((1, D_HID), lambda i: (0, 0)),
                pl.BlockSpec((D_HID, 2 * D_OUT), lambda i: (0, 0)),
                pl.BlockSpec((1, D_OUT), lambda i: (0, 0))],
      out_specs=[pl.BlockSpec((ROW_BLK, 32), lambda i: (i, 0)),
                 pl.BlockSpec((ROW_BLK, 32), lambda i: (i, 0)),
                 pl.BlockSpec((ROW_BLK, D_OUT), lambda i: (i, 0))],
      out_shape=[jax.ShapeDtypeStruct((N_NODES, 32), jnp.float32),
                 jax.ShapeDtypeStruct((N_NODES, 32), jnp.float32),
                 jax.ShapeDtypeStruct((N_NODES, D_OUT), jnp.float32)],
  )(agg_a, agg_b, deg_a, deg_b, z1, bl1, wcat, bl2)


def _tc_out(agg_a, agg_b, deg_a, deg_b, z2):
  def body(a_ref, b_ref, da_ref, db_ref, z2_ref, o_ref):
    deg = da_ref[:, 0:1] + db_ref[:, 0:1]
    inv = 1.0 / jnp.maximum(deg, 1.0)
    o = jnp.concatenate([a_ref[...], b_ref[...]], axis=1) * inv + z2_ref[...]
    m = jnp.max(o, axis=1, keepdims=True)
    e = jnp.exp(o - m)
    o_ref[...] = o - m - jnp.log(jnp.sum(e, axis=1, keepdims=True))

  return pl.pallas_call(
      body,
      grid=(N_NODES // ROW_BLK,),
      in_specs=[pl.BlockSpec((ROW_BLK, 32), lambda i: (i, 0)),
                pl.BlockSpec((ROW_BLK, 32), lambda i: (i, 0)),
                pl.BlockSpec((ROW_BLK, 16), lambda i: (i, 0)),
                pl.BlockSpec((ROW_BLK, 16), lambda i: (i, 0)),
                pl.BlockSpec((ROW_BLK, D_OUT), lambda i: (i, 0))],
      out_specs=pl.BlockSpec((ROW_BLK, D_OUT), lambda i: (i, 0)),
      out_shape=jax.ShapeDtypeStruct((N_NODES, D_OUT), jnp.float32),
  )(agg_a, agg_b, deg_a, deg_b, z2)


_sc_agg64 = _make_sc_aggregate(64, with_deg=True)
_sc_agg32 = _make_sc_aggregate(32, with_deg=False)


def kernel(x, edge_index, Wl1, bl1, Wr1, Wl2, bl2, Wr2):
  src = edge_index[0].astype(jnp.int32)
  dst = edge_index[1].astype(jnp.int32)
  pad = E_PAD - N_EDGES
  src2d = jnp.concatenate([src, jnp.zeros((pad,), jnp.int32)]).reshape(-1, STEP)
  dst2d = jnp.concatenate(
      [dst, jnp.full((pad,), N_NODES, jnp.int32)]).reshape(-1, STEP)

  zf64 = jnp.zeros((N_PAD, 64), jnp.float32)
  zf32 = jnp.zeros((N_PAD, 32), jnp.float32)
  zd = jnp.zeros((N_PAD, 16), jnp.float32)
  ones = jnp.ones((STEP, 16), jnp.float32)

  w1cat = jnp.concatenate([Wl1.T, Wr1.T], axis=1)
  w2cat = jnp.concatenate([Wl2.T, Wr2.T], axis=1)

  y1a, y1b, z1 = _tc_layer1(x, w1cat)
  agg1a, agg1b, dega, degb = _sc_agg64(y1a, y1b, src2d, dst2d, zf64, zd, ones)
  dega = dega[:N_NODES]
  degb = degb[:N_NODES]
  y2a, y2b, z2 = _tc_layer2(agg1a[:N_NODES], agg1b[:N_NODES], dega, degb,
                            z1, bl1.reshape(1, -1), w2cat, bl2.reshape(1, -1))
  agg2a, agg2b = _sc_agg32(y2a, y2b, src2d, dst2d, zf32)
  return _tc_out(agg2a[:N_NODES], agg2b[:N_NODES], dega, degb, z2)


# speedup vs baseline: 6.5436x; 1.3841x over previous
import functools

import jax
import jax.numpy as jnp
from jax import lax
from jax.experimental import pallas as pl
from jax.experimental.pallas import tpu as pltpu
from jax.experimental.pallas import tpu_sc as plsc

N_NODES = 10000
N_EDGES = 320000
D_IN = 128
D_HID = 128
D_OUT = 64

N_TILES = 16
N_PAD = 10112
ROWS_PER_TILE = N_PAD // N_TILES
STEP = 128
S_T = 160
E_PAD = N_TILES * S_T * STEP
DEG_SPLIT = (S_T + 1) // 2
ROW_BLK = 1000


def _make_sc_aggregate(width, with_deg):
  out_type = [jax.ShapeDtypeStruct((N_PAD, width), jnp.float32),
              jax.ShapeDtypeStruct((N_PAD, width), jnp.float32)]
  scratch = [pltpu.VMEM((S_T, STEP), jnp.int32),
             pltpu.VMEM((S_T, STEP), jnp.int32),
             pltpu.VMEM((2, STEP, width), jnp.float32),
             pltpu.VMEM_SHARED((N_PAD, width), jnp.float32),
             pltpu.SemaphoreType.DMA]
  if with_deg:
    out_type += [jax.ShapeDtypeStruct((N_PAD, 16), jnp.float32),
                 jax.ShapeDtypeStruct((N_PAD, 16), jnp.float32)]
    scratch += [pltpu.VMEM((STEP, 16), jnp.float32),
                pltpu.VMEM_SHARED((N_PAD, 16), jnp.float32)]

  mesh = plsc.VectorSubcoreMesh(core_axis_name="c", subcore_axis_name="s")

  def body(*refs):
    if with_deg:
      (ytab, src_hbm, dst_hbm, zf_hbm, zd_hbm, ones_hbm,
       out_a, out_b, deg_a, deg_b,
       src_v, dst_v, rows_v, agg_sh, sem, ones_v, deg_sh) = refs
    else:
      (ytab, src_hbm, dst_hbm, zf_hbm,
       out_a, out_b,
       src_v, dst_v, rows_v, agg_sh, sem) = refs
    c = lax.axis_index("c")
    s = lax.axis_index("s")
    r0 = s * ROWS_PER_TILE
    tile_rows = pl.ds(r0, ROWS_PER_TILE)
    tab = ytab.at[c]

    pltpu.sync_copy(zf_hbm.at[tile_rows], agg_sh.at[tile_rows])
    if with_deg:
      pltpu.sync_copy(zd_hbm.at[tile_rows], deg_sh.at[tile_rows])
      pltpu.sync_copy(ones_hbm, ones_v)
    pltpu.sync_copy(src_hbm.at[pl.ds(s * S_T, S_T)], src_v)
    pltpu.sync_copy(dst_hbm.at[pl.ds(s * S_T, S_T)], dst_v)
    plsc.subcore_barrier()

    pltpu.async_copy(tab.at[src_v.at[0]], rows_v.at[0], sem)

    def step(j, carry):
      @pl.when(j + 1 < S_T)
      def _():
        pltpu.async_copy(tab.at[src_v.at[j + 1]], rows_v.at[(j + 1) % 2], sem)

      buf = rows_v.at[j % 2]
      pltpu.make_async_copy(tab.at[src_v.at[j]], buf, sem).wait()
      pltpu.sync_copy(buf, agg_sh.at[dst_v.at[j]], add=True)
      if with_deg:
        count_here = jnp.where(c == 0, j < DEG_SPLIT, j >= DEG_SPLIT)

        @pl.when(count_here)
        def _():
          pltpu.sync_copy(ones_v, deg_sh.at[dst_v.at[j]], add=True)
      return carry

    lax.fori_loop(0, S_T, step, 0)
    plsc.subcore_barrier()

    @pl.when(c == 0)
    def _():
      pltpu.sync_copy(agg_sh.at[tile_rows], out_a.at[tile_rows])
      if with_deg:
        pltpu.sync_copy(deg_sh.at[tile_rows], deg_a.at[tile_rows])

    @pl.when(c == 1)
    def _():
      pltpu.sync_copy(agg_sh.at[tile_rows], out_b.at[tile_rows])
      if with_deg:
        pltpu.sync_copy(deg_sh.at[tile_rows], deg_b.at[tile_rows])

  return pl.kernel(body, out_type=out_type, mesh=mesh, scratch_types=scratch,
                   compiler_params=pltpu.CompilerParams(use_tc_tiling_on_sc=False))


def _tc_layer1(x, wcat):
  def body(x_ref, w_ref, ytab_ref, z_ref):
    y = jnp.dot(x_ref[...], w_ref[...], preferred_element_type=jnp.float32)
    ytab_ref[0] = y[:, :64]
    ytab_ref[1] = y[:, 64:128]
    z_ref[...] = y[:, 128:]

  return pl.pallas_call(
      body,
      grid=(N_NODES // ROW_BLK,),
      in_specs=[pl.BlockSpec((ROW_BLK, D_IN), lambda i: (i, 0)),
                pl.BlockSpec((D_IN, 2 * D_HID), lambda i: (0, 0))],
      out_specs=[pl.BlockSpec((2, ROW_BLK, 64), lambda i: (0, i, 0)),
                 pl.BlockSpec((ROW_BLK, D_HID), lambda i: (i, 0))],
      out_shape=[jax.ShapeDtypeStruct((2, N_NODES, 64), jnp.float32),
                 jax.ShapeDtypeStruct((N_NODES, D_HID), jnp.float32)],
  )(x, wcat)


def _tc_layer2(agg_a, agg_b, deg_a, deg_b, z1, bl1, wcat, bl2):
  def body(a_ref, b_ref, da_ref, db_ref, z1_ref, bl1_ref, w_ref, bl2_ref,
           ytab_ref, z2_ref):
    deg = da_ref[:, 0:1] + db_ref[:, 0:1]
    inv = 1.0 / jnp.maximum(deg, 1.0)
    mean = jnp.concatenate([a_ref[...], b_ref[...]], axis=1) * inv
    h = jnp.maximum(mean + bl1_ref[...] + z1_ref[...], 0.0)
    y2 = jnp.dot(h, w_ref[...], preferred_element_type=jnp.float32)
    ytab_ref[0] = y2[:, :32]
    ytab_ref[1] = y2[:, 32:64]
    z2_ref[...] = y2[:, 64:] + bl2_ref[...]

  return pl.pallas_call(
      body,
      grid=(N_NODES // ROW_BLK,),
      in_specs=[pl.BlockSpec((ROW_BLK, 64), lambda i: (i, 0)),
                pl.BlockSpec((ROW_BLK, 64), lambda i: (i, 0)),
                pl.BlockSpec((ROW_BLK, 16), lambda i: (i, 0)),
                pl.BlockSpec((ROW_BLK, 16), lambda i: (i, 0)),
                pl.BlockSpec((ROW_BLK, D_HID), lambda i: (i, 0)),
                pl.BlockSpec((1, D_HID), lambda i: (0, 0)),
                pl.BlockSpec((D_HID, 2 * D_OUT), lambda i: (0, 0)),
                pl.BlockSpec((1, D_OUT), lambda i: (0, 0))],
      out_specs=[pl.BlockSpec((2, ROW_BLK, 32), lambda i: (0, i, 0)),
                 pl.BlockSpec((ROW_BLK, D_OUT), lambda i: (i, 0))],
      out_shape=[jax.ShapeDtypeStruct((2, N_NODES, 32), jnp.float32),
                 jax.ShapeDtypeStruct((N_NODES, D_OUT), jnp.float32)],
  )(agg_a, agg_b, deg_a, deg_b, z1, bl1, wcat, bl2)


def _tc_out(agg_a, agg_b, deg_a, deg_b, z2):
  def body(a_ref, b_ref, da_ref, db_ref, z2_ref, o_ref):
    deg = da_ref[:, 0:1] + db_ref[:, 0:1]
    inv = 1.0 / jnp.maximum(deg, 1.0)
    o = jnp.concatenate([a_ref[...], b_ref[...]], axis=1) * inv + z2_ref[...]
    m = jnp.max(o, axis=1, keepdims=True)
    e = jnp.exp(o - m)
    o_ref[...] = o - m - jnp.log(jnp.sum(e, axis=1, keepdims=True))

  return pl.pallas_call(
      body,
      grid=(N_NODES // ROW_BLK,),
      in_specs=[pl.BlockSpec((ROW_BLK, 32), lambda i: (i, 0)),
                pl.BlockSpec((ROW_BLK, 32), lambda i: (i, 0)),
                pl.BlockSpec((ROW_BLK, 16), lambda i: (i, 0)),
                pl.BlockSpec((ROW_BLK, 16), lambda i: (i, 0)),
                pl.BlockSpec((ROW_BLK, D_OUT), lambda i: (i, 0))],
      out_specs=pl.BlockSpec((ROW_BLK, D_OUT), lambda i: (i, 0)),
      out_shape=jax.ShapeDtypeStruct((N_NODES, D_OUT), jnp.float32),
  )(agg_a, agg_b, deg_a, deg_b, z2)


_sc_agg64 = _make_sc_aggregate(64, with_deg=True)
_sc_agg32 = _make_sc_aggregate(32, with_deg=False)


def kernel(x, edge_index, Wl1, bl1, Wr1, Wl2, bl2, Wr2):
  src = edge_index[0].astype(jnp.int32)
  dst = edge_index[1].astype(jnp.int32)
  pad = E_PAD - N_EDGES
  src2d = jnp.concatenate([src, jnp.zeros((pad,), jnp.int32)]).reshape(-1, STEP)
  dst2d = jnp.concatenate(
      [dst, jnp.full((pad,), N_NODES, jnp.int32)]).reshape(-1, STEP)

  zf64 = jnp.zeros((N_PAD, 64), jnp.float32)
  zf32 = jnp.zeros((N_PAD, 32), jnp.float32)
  zd = jnp.zeros((N_PAD, 16), jnp.float32)
  ones = jnp.ones((STEP, 16), jnp.float32)

  w1cat = jnp.concatenate([Wl1.T, Wr1.T], axis=1)
  w2cat = jnp.concatenate([Wl2.T, Wr2.T], axis=1)

  ytab1, z1 = _tc_layer1(x, w1cat)
  agg1a, agg1b, dega, degb = _sc_agg64(ytab1, src2d, dst2d, zf64, zd, ones)
  dega = dega[:N_NODES]
  degb = degb[:N_NODES]
  ytab2, z2 = _tc_layer2(agg1a[:N_NODES], agg1b[:N_NODES], dega, degb,
                         z1, bl1.reshape(1, -1), w2cat, bl2.reshape(1, -1))
  agg2a, agg2b = _sc_agg32(ytab2, src2d, dst2d, zf32)
  return _tc_out(agg2a[:N_NODES], agg2b[:N_NODES], dega, degb, z2)


# speedup vs baseline: 7.1424x; 1.0915x over previous
import functools

import jax
import jax.numpy as jnp
from jax import lax
from jax.experimental import pallas as pl
from jax.experimental.pallas import tpu as pltpu
from jax.experimental.pallas import tpu_sc as plsc

N_NODES = 10000
N_EDGES = 320000
D_IN = 128
D_HID = 128
D_OUT = 64

N_TILES = 16
N_PAD = 10112
ROWS_PER_TILE = N_PAD // N_TILES
STEP = 128
S_T = 160
E_PAD = N_TILES * S_T * STEP
DEG_SPLIT = (S_T + 1) // 2
NBUF = 4
SDEPTH = 2
ROW_BLK = 1000


def _make_sc_aggregate(width, with_deg):
  out_type = [jax.ShapeDtypeStruct((N_PAD, width), jnp.float32),
              jax.ShapeDtypeStruct((N_PAD, width), jnp.float32)]
  scratch = [pltpu.VMEM((S_T, STEP), jnp.int32),
             pltpu.VMEM((S_T, STEP), jnp.int32),
             pltpu.VMEM((NBUF, STEP, width), jnp.float32),
             pltpu.VMEM_SHARED((N_PAD, width), jnp.float32),
             pltpu.SemaphoreType.DMA,
             pltpu.SemaphoreType.DMA]
  if with_deg:
    out_type += [jax.ShapeDtypeStruct((N_PAD, 16), jnp.float32),
                 jax.ShapeDtypeStruct((N_PAD, 16), jnp.float32)]
    scratch += [pltpu.VMEM((STEP, 16), jnp.float32),
                pltpu.VMEM_SHARED((N_PAD, 16), jnp.float32),
                pltpu.SemaphoreType.DMA]

  mesh = plsc.VectorSubcoreMesh(core_axis_name="c", subcore_axis_name="s")

  def body(*refs):
    if with_deg:
      (ytab, src_hbm, dst_hbm, zf_hbm, zd_hbm, ones_hbm,
       out_a, out_b, deg_a, deg_b,
       src_v, dst_v, rows_v, agg_sh, gsem, ssem, ones_v, deg_sh, dsem) = refs
    else:
      (ytab, src_hbm, dst_hbm, zf_hbm,
       out_a, out_b,
       src_v, dst_v, rows_v, agg_sh, gsem, ssem) = refs
    c = lax.axis_index("c")
    s = lax.axis_index("s")
    r0 = s * ROWS_PER_TILE
    tile_rows = pl.ds(r0, ROWS_PER_TILE)
    tab = ytab.at[c]

    pltpu.sync_copy(zf_hbm.at[tile_rows], agg_sh.at[tile_rows])
    if with_deg:
      pltpu.sync_copy(zd_hbm.at[tile_rows], deg_sh.at[tile_rows])
      pltpu.sync_copy(ones_hbm, ones_v)
    pltpu.sync_copy(src_hbm.at[pl.ds(s * S_T, S_T)], src_v)
    pltpu.sync_copy(dst_hbm.at[pl.ds(s * S_T, S_T)], dst_v)
    plsc.subcore_barrier()

    def count_here(j):
      return jnp.where(c == 0, j < DEG_SPLIT, j >= DEG_SPLIT)

    for b in range(SDEPTH):
      pltpu.async_copy(tab.at[src_v.at[b]], rows_v.at[b], gsem)

    def step(j, carry):
      buf = rows_v.at[j % NBUF]
      pltpu.make_async_copy(tab.at[src_v.at[j]], buf, gsem).wait()
      pltpu.async_copy(buf, agg_sh.at[dst_v.at[j]], ssem, add=True)
      if with_deg:
        @pl.when(count_here(j))
        def _():
          pltpu.async_copy(ones_v, deg_sh.at[dst_v.at[j]], dsem, add=True)

      @pl.when(j >= SDEPTH)
      def _():
        pltpu.make_async_copy(rows_v.at[(j - SDEPTH) % NBUF],
                              agg_sh.at[dst_v.at[j - SDEPTH]], ssem).wait()
        if with_deg:
          @pl.when(count_here(j - SDEPTH))
          def _():
            pltpu.make_async_copy(ones_v, deg_sh.at[dst_v.at[j - SDEPTH]],
                                  dsem).wait()

      @pl.when(j + SDEPTH < S_T)
      def _():
        pltpu.async_copy(tab.at[src_v.at[j + SDEPTH]],
                         rows_v.at[(j + SDEPTH) % NBUF], gsem)
      return carry

    lax.fori_loop(0, S_T, step, 0)
    for j in range(S_T - SDEPTH, S_T):
      pltpu.make_async_copy(rows_v.at[j % NBUF],
                            agg_sh.at[dst_v.at[j]], ssem).wait()
      if with_deg:
        @pl.when(count_here(j))
        def _():
          pltpu.make_async_copy(ones_v, deg_sh.at[dst_v.at[j]], dsem).wait()
    plsc.subcore_barrier()

    @pl.when(c == 0)
    def _():
      pltpu.sync_copy(agg_sh.at[tile_rows], out_a.at[tile_rows])
      if with_deg:
        pltpu.sync_copy(deg_sh.at[tile_rows], deg_a.at[tile_rows])

    @pl.when(c == 1)
    def _():
      pltpu.sync_copy(agg_sh.at[tile_rows], out_b.at[tile_rows])
      if with_deg:
        pltpu.sync_copy(deg_sh.at[tile_rows], deg_b.at[tile_rows])

  return pl.kernel(body, out_type=out_type, mesh=mesh, scratch_types=scratch,
                   compiler_params=pltpu.CompilerParams(use_tc_tiling_on_sc=False))


def _tc_layer1(x, wcat):
  def body(x_ref, w_ref, ytab_ref, z_ref):
    y = jnp.dot(x_ref[...], w_ref[...], preferred_element_type=jnp.float32)
    ytab_ref[0] = y[:, :64]
    ytab_ref[1] = y[:, 64:128]
    z_ref[...] = y[:, 128:]

  return pl.pallas_call(
      body,
      grid=(N_NODES // ROW_BLK,),
      in_specs=[pl.BlockSpec((ROW_BLK, D_IN), lambda i: (i, 0)),
                pl.BlockSpec((D_IN, 2 * D_HID), lambda i: (0, 0))],
      out_specs=[pl.BlockSpec((2, ROW_BLK, 64), lambda i: (0, i, 0)),
                 pl.BlockSpec((ROW_BLK, D_HID), lambda i: (i, 0))],
      out_shape=[jax.ShapeDtypeStruct((2, N_NODES, 64), jnp.float32),
                 jax.ShapeDtypeStruct((N_NODES, D_HID), jnp.float32)],
  )(x, wcat)


def _tc_layer2(agg_a, agg_b, deg_a, deg_b, z1, bl1, wcat, bl2):
  def body(a_ref, b_ref, da_ref, db_ref, z1_ref, bl1_ref, w_ref, bl2_ref,
           ytab_ref, z2_ref):
    deg = da_ref[:, 0:1] + db_ref[:, 0:1]
    inv = 1.0 / jnp.maximum(deg, 1.0)
    mean = jnp.concatenate([a_ref[...], b_ref[...]], axis=1) * inv
    h = jnp.maximum(mean + bl1_ref[...] + z1_ref[...], 0.0)
    y2 = jnp.dot(h, w_ref[...], preferred_element_type=jnp.float32)
    ytab_ref[0] = y2[:, :32]
    ytab_ref[1] = y2[:, 32:64]
    z2_ref[...] = y2[:, 64:] + bl2_ref[...]

  return pl.pallas_call(
      body,
      grid=(N_NODES // ROW_BLK,),
      in_specs=[pl.BlockSpec((ROW_BLK, 64), lambda i: (i, 0)),
                pl.BlockSpec((ROW_BLK, 64), lambda i: (i, 0)),
                pl.BlockSpec((ROW_BLK, 16), lambda i: (i, 0)),
                pl.BlockSpec((ROW_BLK, 16), lambda i: (i, 0)),
                pl.BlockSpec((ROW_BLK, D_HID), lambda i: (i, 0)),
                pl.BlockSpec((1, D_HID), lambda i: (0, 0)),
                pl.BlockSpec((D_HID, 2 * D_OUT), lambda i: (0, 0)),
                pl.BlockSpec((1, D_OUT), lambda i: (0, 0))],
      out_specs=[pl.BlockSpec((2, ROW_BLK, 32), lambda i: (0, i, 0)),
                 pl.BlockSpec((ROW_BLK, D_OUT), lambda i: (i, 0))],
      out_shape=[jax.ShapeDtypeStruct((2, N_NODES, 32), jnp.float32),
                 jax.ShapeDtypeStruct((N_NODES, D_OUT), jnp.float32)],
  )(agg_a, agg_b, deg_a, deg_b, z1, bl1, wcat, bl2)


def _tc_out(agg_a, agg_b, deg_a, deg_b, z2):
  def body(a_ref, b_ref, da_ref, db_ref, z2_ref, o_ref):
    deg = da_ref[:, 0:1] + db_ref[:, 0:1]
    inv = 1.0 / jnp.maximum(deg, 1.0)
    o = jnp.concatenate([a_ref[...], b_ref[...]], axis=1) * inv + z2_ref[...]
    m = jnp.max(o, axis=1, keepdims=True)
    e = jnp.exp(o - m)
    o_ref[...] = o - m - jnp.log(jnp.sum(e, axis=1, keepdims=True))

  return pl.pallas_call(
      body,
      grid=(N_NODES // ROW_BLK,),
      in_specs=[pl.BlockSpec((ROW_BLK, 32), lambda i: (i, 0)),
                pl.BlockSpec((ROW_BLK, 32), lambda i: (i, 0)),
                pl.BlockSpec((ROW_BLK, 16), lambda i: (i, 0)),
                pl.BlockSpec((ROW_BLK, 16), lambda i: (i, 0)),
                pl.BlockSpec((ROW_BLK, D_OUT), lambda i: (i, 0))],
      out_specs=pl.BlockSpec((ROW_BLK, D_OUT), lambda i: (i, 0)),
      out_shape=jax.ShapeDtypeStruct((N_NODES, D_OUT), jnp.float32),
  )(agg_a, agg_b, deg_a, deg_b, z2)


_sc_agg64 = _make_sc_aggregate(64, with_deg=True)
_sc_agg32 = _make_sc_aggregate(32, with_deg=False)


def kernel(x, edge_index, Wl1, bl1, Wr1, Wl2, bl2, Wr2):
  src = edge_index[0].astype(jnp.int32)
  dst = edge_index[1].astype(jnp.int32)
  pad = E_PAD - N_EDGES
  src2d = jnp.concatenate([src, jnp.zeros((pad,), jnp.int32)]).reshape(-1, STEP)
  dst2d = jnp.concatenate(
      [dst, jnp.full((pad,), N_NODES, jnp.int32)]).reshape(-1, STEP)

  zf64 = jnp.zeros((N_PAD, 64), jnp.float32)
  zf32 = jnp.zeros((N_PAD, 32), jnp.float32)
  zd = jnp.zeros((N_PAD, 16), jnp.float32)
  ones = jnp.ones((STEP, 16), jnp.float32)

  w1cat = jnp.concatenate([Wl1.T, Wr1.T], axis=1)
  w2cat = jnp.concatenate([Wl2.T, Wr2.T], axis=1)

  ytab1, z1 = _tc_layer1(x, w1cat)
  agg1a, agg1b, dega, degb = _sc_agg64(ytab1, src2d, dst2d, zf64, zd, ones)
  ytab2, z2 = _tc_layer2(agg1a, agg1b, dega, degb,
                         z1, bl1.reshape(1, -1), w2cat, bl2.reshape(1, -1))
  agg2a, agg2b = _sc_agg32(ytab2, src2d, dst2d, zf32)
  return _tc_out(agg2a, agg2b, dega, degb, z2)
